# Initial kernel scaffold; baseline (speedup 1.0000x reference)
import jax
import jax.numpy as jnp
from jax.experimental import pallas as pl


def kernel(x, edge_index, batch, W1_0, b1_0, W2_0, b2_0, W1_1, b1_1, W2_1, b2_1, Wfc, bfc):
    raise NotImplementedError("write your pallas kernel here")


# speedup vs baseline: 2.5323x; 2.5323x over previous
import functools

import jax
import jax.numpy as jnp
from jax import lax
from jax.experimental import pallas as pl
from jax.experimental.pallas import tpu as pltpu
from jax.experimental.pallas import tpu_sc as plsc

_N = 10000
_E = 320000
_D = 128
_G = 64
_NC = 2
_NS = 16
_NW = _NC * _NS
_CH = 128
_EPT = 10240
_EPAD = _NW * _EPT
_RPT = 624
_LAST = _N - _RPT * (_NS - 1)
_PADROWS = 16


def _spmm_body(x_hbm, src_hbm, dst_hbm, zero_hbm, out_hbm,
               idx_s, idx_d, rows, agg, sem_g):
    cid = lax.axis_index("c")
    sid = lax.axis_index("s")
    wid = cid * _NS + sid
    base = sid * _RPT

    @pl.when(sid < _NS - 1)
    def _():
        pltpu.sync_copy(zero_hbm.at[pl.ds(base, _RPT)],
                        agg.at[pl.ds(base, _RPT)])

    @pl.when(sid == _NS - 1)
    def _():
        pltpu.sync_copy(zero_hbm.at[pl.ds(_RPT * (_NS - 1), _LAST + _PADROWS)],
                        agg.at[pl.ds(_RPT * (_NS - 1), _LAST + _PADROWS)])

    plsc.subcore_barrier()

    ebase = wid * _EPT

    def chunk(k, carry):
        off = ebase + k * _CH
        pltpu.sync_copy(src_hbm.at[pl.ds(off, _CH)], idx_s)
        pltpu.sync_copy(dst_hbm.at[pl.ds(off, _CH)], idx_d)
        pltpu.async_copy(x_hbm.at[idx_s], rows, sem_g).wait()
        pltpu.sync_copy(rows, agg.at[idx_d], add=True)
        return carry

    lax.fori_loop(0, _EPT // _CH, chunk, 0)

    plsc.subcore_barrier()

    @pl.when(jnp.logical_and(cid == 0, sid < _NS - 1))
    def _():
        pltpu.sync_copy(agg.at[pl.ds(base, _RPT)],
                        out_hbm.at[0].at[pl.ds(base, _RPT)])

    @pl.when(jnp.logical_and(cid == 0, sid == _NS - 1))
    def _():
        pltpu.sync_copy(agg.at[pl.ds(_RPT * (_NS - 1), _LAST)],
                        out_hbm.at[0].at[pl.ds(_RPT * (_NS - 1), _LAST)])

    @pl.when(jnp.logical_and(cid == 1, sid < _NS - 1))
    def _():
        pltpu.sync_copy(agg.at[pl.ds(base, _RPT)],
                        out_hbm.at[1].at[pl.ds(base, _RPT)])

    @pl.when(jnp.logical_and(cid == 1, sid == _NS - 1))
    def _():
        pltpu.sync_copy(agg.at[pl.ds(_RPT * (_NS - 1), _LAST)],
                        out_hbm.at[1].at[pl.ds(_RPT * (_NS - 1), _LAST)])


@functools.cache
def _get_spmm_call():
    return pl.kernel(
        _spmm_body,
        out_type=jax.ShapeDtypeStruct((_NC, _N, _D), jnp.float32),
        mesh=plsc.VectorSubcoreMesh(core_axis_name="c", subcore_axis_name="s",
                                    num_cores=_NC, num_subcores=_NS),
        scratch_types=[
            pltpu.VMEM((_CH,), jnp.int32),
            pltpu.VMEM((_CH,), jnp.int32),
            pltpu.VMEM((_CH, _D), jnp.float32),
            pltpu.VMEM_SHARED((_N + _PADROWS, _D), jnp.float32),
            pltpu.SemaphoreType.DMA,
        ],
    )


def _spmm_call(x, srcp, dstp, zero_rows):
    return _get_spmm_call()(x, srcp, dstp, zero_rows)


def _mlp_body(x_ref, a0_ref, a1_ref, w1_ref, b1_ref, w2_ref, b2_ref, o_ref):
    h = x_ref[...] + a0_ref[...] + a1_ref[...]
    h = jnp.maximum(
        jnp.dot(h, w1_ref[...], preferred_element_type=jnp.float32)
        + b1_ref[...], 0.0)
    h = jnp.maximum(
        jnp.dot(h, w2_ref[...], preferred_element_type=jnp.float32)
        + b2_ref[...], 0.0)
    o_ref[...] = h


_BR = 2000
_NB = _N // _BR


def _mlp_tc(x, a0, a1, w1, b1, w2, b2):
    row_spec = pl.BlockSpec((_BR, _D), lambda i: (i, 0))
    w_spec = pl.BlockSpec((_D, _D), lambda i: (0, 0))
    b_spec = pl.BlockSpec((1, _D), lambda i: (0, 0))
    return pl.pallas_call(
        _mlp_body,
        grid=(_NB,),
        in_specs=[row_spec, row_spec, row_spec, w_spec, b_spec, w_spec, b_spec],
        out_specs=row_spec,
        out_shape=jax.ShapeDtypeStruct((_N, _D), jnp.float32),
    )(x, a0, a1, w1, b1, w2, b2)


def _final_body(x_ref, a0_ref, a1_ref, w1_ref, b1_ref, w2_ref, b2_ref,
                bt_ref, wfc_ref, bfc_ref, o_ref, p1_ref, p2_ref):
    i = pl.program_id(0)

    @pl.when(i == 0)
    def _():
        p1_ref[...] = jnp.zeros_like(p1_ref)
        p2_ref[...] = jnp.zeros_like(p2_ref)

    x1b = x_ref[...]
    h = x1b + a0_ref[...] + a1_ref[...]
    h = jnp.maximum(
        jnp.dot(h, w1_ref[...], preferred_element_type=jnp.float32)
        + b1_ref[...], 0.0)
    x2b = jnp.maximum(
        jnp.dot(h, w2_ref[...], preferred_element_type=jnp.float32)
        + b2_ref[...], 0.0)

    b2d = bt_ref[...].reshape(1, _BR)
    gids = lax.broadcasted_iota(jnp.int32, (_G, _BR), 0)
    pt = (gids == b2d).astype(jnp.float32)
    p1_ref[...] += jnp.dot(pt, x1b, preferred_element_type=jnp.float32)
    p2_ref[...] += jnp.dot(pt, x2b, preferred_element_type=jnp.float32)

    @pl.when(i == _NB - 1)
    def _():
        pooled = (
            jnp.dot(p1_ref[...], wfc_ref[0:_D, :],
                    preferred_element_type=jnp.float32)
            + jnp.dot(p2_ref[...], wfc_ref[_D:2 * _D, :],
                      preferred_element_type=jnp.float32)
            + bfc_ref[...])
        m = jnp.max(pooled, axis=-1, keepdims=True)
        lse = jnp.log(jnp.sum(jnp.exp(pooled - m), axis=-1, keepdims=True)) + m
        o_ref[...] = pooled - lse


def _final_tc(x1, a0, a1, w1, b1, w2, b2, batch3, wfc, bfc):
    row_spec = pl.BlockSpec((_BR, _D), lambda i: (i, 0))
    w_spec = pl.BlockSpec((_D, _D), lambda i: (0, 0))
    b_spec = pl.BlockSpec((1, _D), lambda i: (0, 0))
    return pl.pallas_call(
        _final_body,
        grid=(_NB,),
        in_specs=[
            row_spec, row_spec, row_spec, w_spec, b_spec, w_spec, b_spec,
            pl.BlockSpec((1, 1, _BR), lambda i: (i, 0, 0)),
            pl.BlockSpec((2 * _D, _D), lambda i: (0, 0)),
            b_spec,
        ],
        out_specs=pl.BlockSpec((_G, _D), lambda i: (0, 0)),
        out_shape=jax.ShapeDtypeStruct((_G, _D), jnp.float32),
        scratch_shapes=[
            pltpu.VMEM((_G, _D), jnp.float32),
            pltpu.VMEM((_G, _D), jnp.float32),
        ],
    )(x1, a0, a1, w1, b1, w2, b2, batch3, wfc, bfc)


def kernel(x, edge_index, batch, W1_0, b1_0, W2_0, b2_0,
           W1_1, b1_1, W2_1, b2_1, Wfc, bfc):
    src = edge_index[0]
    dst = edge_index[1]
    npad = _EPAD - _E
    srcp = jnp.concatenate([src, jnp.zeros((npad,), jnp.int32)])
    dstp = jnp.concatenate([dst, jnp.full((npad,), _N, jnp.int32)])
    zero_rows = jnp.zeros((_N + _PADROWS, _D), jnp.float32)

    b1_0r = b1_0.reshape(1, _D)
    b2_0r = b2_0.reshape(1, _D)
    b1_1r = b1_1.reshape(1, _D)
    b2_1r = b2_1.reshape(1, _D)
    bfcr = bfc.reshape(1, _D)
    batch3 = batch.reshape(_NB, 1, _BR)

    agg1 = _spmm_call(x, srcp, dstp, zero_rows)
    x1 = _mlp_tc(x, agg1[0], agg1[1], W1_0, b1_0r, W2_0, b2_0r)
    agg2 = _spmm_call(x1, srcp, dstp, zero_rows)
    out = _final_tc(x1, agg2[0], agg2[1], W1_1, b1_1r, W2_1, b2_1r,
                    batch3, Wfc, bfcr)
    return out


# speedup vs baseline: 3.3913x; 1.3392x over previous
import functools

import jax
import jax.numpy as jnp
from jax import lax
from jax.experimental import pallas as pl
from jax.experimental.pallas import tpu as pltpu
from jax.experimental.pallas import tpu_sc as plsc

_N = 10000
_E = 320000
_D = 128
_G = 64
_NC = 2
_NS = 16
_NW = _NC * _NS
_CH = 128
_EPT = 10240
_EPAD = _NW * _EPT
_RPT = 624
_LAST = _N - _RPT * (_NS - 1)
_PADROWS = 16


_NCHUNK = _EPT // _CH
_HCHUNK = _NCHUNK // 2


def _spmm_body(x_hbm, src_hbm, dst_hbm, zero_hbm, out_hbm,
               sidx, didx, rows0, rows1, agg,
               semg0, semg1, sems0, sems1):
    cid = lax.axis_index("c")
    sid = lax.axis_index("s")
    wid = cid * _NS + sid
    base = sid * _RPT

    @pl.when(sid < _NS - 1)
    def _():
        pltpu.sync_copy(zero_hbm.at[pl.ds(base, _RPT)],
                        agg.at[pl.ds(base, _RPT)])

    @pl.when(sid == _NS - 1)
    def _():
        pltpu.sync_copy(zero_hbm.at[pl.ds(_RPT * (_NS - 1), _LAST + _PADROWS)],
                        agg.at[pl.ds(_RPT * (_NS - 1), _LAST + _PADROWS)])

    plsc.subcore_barrier()

    def gather(k, rows, semg):
        pltpu.async_copy(x_hbm.at[sidx.at[k]], rows, semg)

    def gather_wait(rows, semg):
        pltpu.make_async_copy(x_hbm.at[sidx.at[0]], rows, semg).wait()

    def scatter(k, rows, sems):
        pltpu.async_copy(rows, agg.at[didx.at[k]], sems, add=True)

    def scatter_wait(rows, sems):
        pltpu.make_async_copy(rows, agg.at[didx.at[0]], sems).wait()

    for h in range(2):
        hbase = wid * _NCHUNK + h * _HCHUNK
        pltpu.sync_copy(src_hbm.at[pl.ds(hbase, _HCHUNK)], sidx)
        pltpu.sync_copy(dst_hbm.at[pl.ds(hbase, _HCHUNK)], didx)
        gather(0, rows0, semg0)

        def step(i, carry):
            k0 = 2 * i
            gather_wait(rows0, semg0)

            @pl.when(i > 0)
            def _():
                scatter_wait(rows1, sems1)

            gather(k0 + 1, rows1, semg1)
            scatter(k0, rows0, sems0)
            gather_wait(rows1, semg1)
            scatter_wait(rows0, sems0)

            @pl.when(i < _HCHUNK // 2 - 1)
            def _():
                gather(k0 + 2, rows0, semg0)

            scatter(k0 + 1, rows1, sems1)
            return carry

        lax.fori_loop(0, _HCHUNK // 2, step, 0)
        scatter_wait(rows1, sems1)

    plsc.subcore_barrier()

    @pl.when(jnp.logical_and(cid == 0, sid < _NS - 1))
    def _():
        pltpu.sync_copy(agg.at[pl.ds(base, _RPT)],
                        out_hbm.at[0].at[pl.ds(base, _RPT)])

    @pl.when(jnp.logical_and(cid == 0, sid == _NS - 1))
    def _():
        pltpu.sync_copy(agg.at[pl.ds(_RPT * (_NS - 1), _LAST)],
                        out_hbm.at[0].at[pl.ds(_RPT * (_NS - 1), _LAST)])

    @pl.when(jnp.logical_and(cid == 1, sid < _NS - 1))
    def _():
        pltpu.sync_copy(agg.at[pl.ds(base, _RPT)],
                        out_hbm.at[1].at[pl.ds(base, _RPT)])

    @pl.when(jnp.logical_and(cid == 1, sid == _NS - 1))
    def _():
        pltpu.sync_copy(agg.at[pl.ds(_RPT * (_NS - 1), _LAST)],
                        out_hbm.at[1].at[pl.ds(_RPT * (_NS - 1), _LAST)])


@functools.cache
def _get_spmm_call():
    return pl.kernel(
        _spmm_body,
        out_type=jax.ShapeDtypeStruct((_NC, _N, _D), jnp.float32),
        mesh=plsc.VectorSubcoreMesh(core_axis_name="c", subcore_axis_name="s",
                                    num_cores=_NC, num_subcores=_NS),
        scratch_types=[
            pltpu.VMEM((_HCHUNK, _CH), jnp.int32),
            pltpu.VMEM((_HCHUNK, _CH), jnp.int32),
            pltpu.VMEM((_CH, _D), jnp.float32),
            pltpu.VMEM((_CH, _D), jnp.float32),
            pltpu.VMEM_SHARED((_N + _PADROWS, _D), jnp.float32),
            pltpu.SemaphoreType.DMA,
            pltpu.SemaphoreType.DMA,
            pltpu.SemaphoreType.DMA,
            pltpu.SemaphoreType.DMA,
        ],
    )


def _spmm_call(x, srcp, dstp, zero_rows):
    return _get_spmm_call()(x, srcp, dstp, zero_rows)


def _mlp_body(x_ref, a0_ref, a1_ref, w1_ref, b1_ref, w2_ref, b2_ref, o_ref):
    h = x_ref[...] + a0_ref[...] + a1_ref[...]
    h = jnp.maximum(
        jnp.dot(h, w1_ref[...], preferred_element_type=jnp.float32)
        + b1_ref[...], 0.0)
    h = jnp.maximum(
        jnp.dot(h, w2_ref[...], preferred_element_type=jnp.float32)
        + b2_ref[...], 0.0)
    o_ref[...] = h


_BR = 2000
_NB = _N // _BR


def _mlp_tc(x, a0, a1, w1, b1, w2, b2):
    row_spec = pl.BlockSpec((_BR, _D), lambda i: (i, 0))
    w_spec = pl.BlockSpec((_D, _D), lambda i: (0, 0))
    b_spec = pl.BlockSpec((1, _D), lambda i: (0, 0))
    return pl.pallas_call(
        _mlp_body,
        grid=(_NB,),
        in_specs=[row_spec, row_spec, row_spec, w_spec, b_spec, w_spec, b_spec],
        out_specs=row_spec,
        out_shape=jax.ShapeDtypeStruct((_N, _D), jnp.float32),
    )(x, a0, a1, w1, b1, w2, b2)


def _final_body(x_ref, a0_ref, a1_ref, w1_ref, b1_ref, w2_ref, b2_ref,
                bt_ref, wfc_ref, bfc_ref, o_ref, p1_ref, p2_ref):
    i = pl.program_id(0)

    @pl.when(i == 0)
    def _():
        p1_ref[...] = jnp.zeros_like(p1_ref)
        p2_ref[...] = jnp.zeros_like(p2_ref)

    x1b = x_ref[...]
    h = x1b + a0_ref[...] + a1_ref[...]
    h = jnp.maximum(
        jnp.dot(h, w1_ref[...], preferred_element_type=jnp.float32)
        + b1_ref[...], 0.0)
    x2b = jnp.maximum(
        jnp.dot(h, w2_ref[...], preferred_element_type=jnp.float32)
        + b2_ref[...], 0.0)

    b2d = bt_ref[...].reshape(1, _BR)
    gids = lax.broadcasted_iota(jnp.int32, (_G, _BR), 0)
    pt = (gids == b2d).astype(jnp.float32)
    p1_ref[...] += jnp.dot(pt, x1b, preferred_element_type=jnp.float32)
    p2_ref[...] += jnp.dot(pt, x2b, preferred_element_type=jnp.float32)

    @pl.when(i == _NB - 1)
    def _():
        pooled = (
            jnp.dot(p1_ref[...], wfc_ref[0:_D, :],
                    preferred_element_type=jnp.float32)
            + jnp.dot(p2_ref[...], wfc_ref[_D:2 * _D, :],
                      preferred_element_type=jnp.float32)
            + bfc_ref[...])
        m = jnp.max(pooled, axis=-1, keepdims=True)
        lse = jnp.log(jnp.sum(jnp.exp(pooled - m), axis=-1, keepdims=True)) + m
        o_ref[...] = pooled - lse


def _final_tc(x1, a0, a1, w1, b1, w2, b2, batch3, wfc, bfc):
    row_spec = pl.BlockSpec((_BR, _D), lambda i: (i, 0))
    w_spec = pl.BlockSpec((_D, _D), lambda i: (0, 0))
    b_spec = pl.BlockSpec((1, _D), lambda i: (0, 0))
    return pl.pallas_call(
        _final_body,
        grid=(_NB,),
        in_specs=[
            row_spec, row_spec, row_spec, w_spec, b_spec, w_spec, b_spec,
            pl.BlockSpec((1, 1, _BR), lambda i: (i, 0, 0)),
            pl.BlockSpec((2 * _D, _D), lambda i: (0, 0)),
            b_spec,
        ],
        out_specs=pl.BlockSpec((_G, _D), lambda i: (0, 0)),
        out_shape=jax.ShapeDtypeStruct((_G, _D), jnp.float32),
        scratch_shapes=[
            pltpu.VMEM((_G, _D), jnp.float32),
            pltpu.VMEM((_G, _D), jnp.float32),
        ],
    )(x1, a0, a1, w1, b1, w2, b2, batch3, wfc, bfc)


def kernel(x, edge_index, batch, W1_0, b1_0, W2_0, b2_0,
           W1_1, b1_1, W2_1, b2_1, Wfc, bfc):
    src = edge_index[0]
    dst = edge_index[1]
    npad = _EPAD - _E
    srcp = jnp.concatenate([src, jnp.zeros((npad,), jnp.int32)])
    dstp = jnp.concatenate([dst, jnp.full((npad,), _N, jnp.int32)])
    srcp = srcp.reshape(_NW * _NCHUNK, _CH)
    dstp = dstp.reshape(_NW * _NCHUNK, _CH)
    zero_rows = jnp.zeros((_N + _PADROWS, _D), jnp.float32)

    b1_0r = b1_0.reshape(1, _D)
    b2_0r = b2_0.reshape(1, _D)
    b1_1r = b1_1.reshape(1, _D)
    b2_1r = b2_1.reshape(1, _D)
    bfcr = bfc.reshape(1, _D)
    batch3 = batch.reshape(_NB, 1, _BR)

    agg1 = _spmm_call(x, srcp, dstp, zero_rows)
    x1 = _mlp_tc(x, agg1[0], agg1[1], W1_0, b1_0r, W2_0, b2_0r)
    agg2 = _spmm_call(x1, srcp, dstp, zero_rows)
    out = _final_tc(x1, agg2[0], agg2[1], W1_1, b1_1r, W2_1, b2_1r,
                    batch3, Wfc, bfcr)
    return out


# speedup vs baseline: 9.8056x; 2.8914x over previous
import functools

import jax
import jax.numpy as jnp
from jax import lax
from jax.experimental import pallas as pl
from jax.experimental.pallas import tpu as pltpu
from jax.experimental.pallas import tpu_sc as plsc

_N = 10000
_E = 320000
_D = 128
_G = 64
_NC = 2
_NS = 16
_NW = _NC * _NS
_CH = 125
_EPT = 10000
_RPT = 624
_LAST = _N - _RPT * (_NS - 1)
_PADROWS = 16


_NCHUNK = _EPT // _CH
_HCHUNK = _NCHUNK // 2


def _spmm_body(x_hbm, src_hbm, dst_hbm, zero_hbm, out_hbm,
               sidx, didx, rows0, rows1, agg,
               semg0, semg1, sems0, sems1):
    cid = lax.axis_index("c")
    sid = lax.axis_index("s")
    wid = cid * _NS + sid
    base = sid * _RPT

    @pl.when(sid < _NS - 1)
    def _():
        pltpu.sync_copy(zero_hbm.at[pl.ds(base, _RPT)],
                        agg.at[pl.ds(base, _RPT)])

    @pl.when(sid == _NS - 1)
    def _():
        pltpu.sync_copy(zero_hbm.at[pl.ds(_RPT * (_NS - 1), _LAST + _PADROWS)],
                        agg.at[pl.ds(_RPT * (_NS - 1), _LAST + _PADROWS)])

    plsc.subcore_barrier()

    def gather(k, rows, semg):
        pltpu.async_copy(x_hbm.at[sidx.at[k]], rows, semg)

    def gather_wait(rows, semg):
        pltpu.make_async_copy(x_hbm.at[sidx.at[0]], rows, semg).wait()

    def scatter(k, rows, sems):
        pltpu.async_copy(rows, agg.at[didx.at[k]], sems, add=True)

    def scatter_wait(rows, sems):
        pltpu.make_async_copy(rows, agg.at[didx.at[0]], sems).wait()

    for h in range(2):
        hbase = wid * _NCHUNK + h * _HCHUNK
        pltpu.sync_copy(src_hbm.at[pl.ds(hbase, _HCHUNK)], sidx)
        pltpu.sync_copy(dst_hbm.at[pl.ds(hbase, _HCHUNK)], didx)
        gather(0, rows0, semg0)

        def step(i, carry):
            k0 = 2 * i
            gather_wait(rows0, semg0)

            @pl.when(i > 0)
            def _():
                scatter_wait(rows1, sems1)

            gather(k0 + 1, rows1, semg1)
            scatter(k0, rows0, sems0)
            gather_wait(rows1, semg1)
            scatter_wait(rows0, sems0)

            @pl.when(i < _HCHUNK // 2 - 1)
            def _():
                gather(k0 + 2, rows0, semg0)

            scatter(k0 + 1, rows1, sems1)
            return carry

        lax.fori_loop(0, _HCHUNK // 2, step, 0)
        scatter_wait(rows1, sems1)

    plsc.subcore_barrier()

    @pl.when(jnp.logical_and(cid == 0, sid < _NS - 1))
    def _():
        pltpu.sync_copy(agg.at[pl.ds(base, _RPT)],
                        out_hbm.at[0].at[pl.ds(base, _RPT)])

    @pl.when(jnp.logical_and(cid == 0, sid == _NS - 1))
    def _():
        pltpu.sync_copy(agg.at[pl.ds(_RPT * (_NS - 1), _LAST)],
                        out_hbm.at[0].at[pl.ds(_RPT * (_NS - 1), _LAST)])

    @pl.when(jnp.logical_and(cid == 1, sid < _NS - 1))
    def _():
        pltpu.sync_copy(agg.at[pl.ds(base, _RPT)],
                        out_hbm.at[1].at[pl.ds(base, _RPT)])

    @pl.when(jnp.logical_and(cid == 1, sid == _NS - 1))
    def _():
        pltpu.sync_copy(agg.at[pl.ds(_RPT * (_NS - 1), _LAST)],
                        out_hbm.at[1].at[pl.ds(_RPT * (_NS - 1), _LAST)])


@functools.cache
def _get_spmm_call():
    return pl.kernel(
        _spmm_body,
        out_type=jax.ShapeDtypeStruct((_NC, _N, _D), jnp.float32),
        mesh=plsc.VectorSubcoreMesh(core_axis_name="c", subcore_axis_name="s",
                                    num_cores=_NC, num_subcores=_NS),
        scratch_types=[
            pltpu.VMEM((_HCHUNK, _CH), jnp.int32),
            pltpu.VMEM((_HCHUNK, _CH), jnp.int32),
            pltpu.VMEM((_CH, _D), jnp.float32),
            pltpu.VMEM((_CH, _D), jnp.float32),
            pltpu.VMEM_SHARED((_N + _PADROWS, _D), jnp.float32),
            pltpu.SemaphoreType.DMA,
            pltpu.SemaphoreType.DMA,
            pltpu.SemaphoreType.DMA,
            pltpu.SemaphoreType.DMA,
        ],
    )


def _spmm_call(x, srcp, dstp, zero_rows):
    return _get_spmm_call()(x, srcp, dstp, zero_rows)


def _mlp_body(x_ref, a0_ref, a1_ref, w1_ref, b1_ref, w2_ref, b2_ref, o_ref):
    h = x_ref[...] + a0_ref[...] + a1_ref[...]
    h = jnp.maximum(
        jnp.dot(h, w1_ref[...], preferred_element_type=jnp.float32)
        + b1_ref[...], 0.0)
    h = jnp.maximum(
        jnp.dot(h, w2_ref[...], preferred_element_type=jnp.float32)
        + b2_ref[...], 0.0)
    o_ref[...] = h


_BR = 2000
_NB = _N // _BR


def _mlp_tc(x, a0, a1, w1, b1, w2, b2):
    row_spec = pl.BlockSpec((_BR, _D), lambda i: (i, 0))
    w_spec = pl.BlockSpec((_D, _D), lambda i: (0, 0))
    b_spec = pl.BlockSpec((1, _D), lambda i: (0, 0))
    return pl.pallas_call(
        _mlp_body,
        grid=(_NB,),
        in_specs=[row_spec, row_spec, row_spec, w_spec, b_spec, w_spec, b_spec],
        out_specs=row_spec,
        out_shape=jax.ShapeDtypeStruct((_N, _D), jnp.float32),
    )(x, a0, a1, w1, b1, w2, b2)


def _final_body(x_ref, a0_ref, a1_ref, w1_ref, b1_ref, w2_ref, b2_ref,
                bt_ref, wfc_ref, bfc_ref, o_ref, p1_ref, p2_ref):
    i = pl.program_id(0)

    @pl.when(i == 0)
    def _():
        p1_ref[...] = jnp.zeros_like(p1_ref)
        p2_ref[...] = jnp.zeros_like(p2_ref)

    x1b = x_ref[...]
    h = x1b + a0_ref[...] + a1_ref[...]
    h = jnp.maximum(
        jnp.dot(h, w1_ref[...], preferred_element_type=jnp.float32)
        + b1_ref[...], 0.0)
    x2b = jnp.maximum(
        jnp.dot(h, w2_ref[...], preferred_element_type=jnp.float32)
        + b2_ref[...], 0.0)

    b2d = bt_ref[...].reshape(1, _BR)
    gids = lax.broadcasted_iota(jnp.int32, (_G, _BR), 0)
    pt = (gids == b2d).astype(jnp.float32)
    p1_ref[...] += jnp.dot(pt, x1b, preferred_element_type=jnp.float32)
    p2_ref[...] += jnp.dot(pt, x2b, preferred_element_type=jnp.float32)

    @pl.when(i == _NB - 1)
    def _():
        pooled = (
            jnp.dot(p1_ref[...], wfc_ref[0:_D, :],
                    preferred_element_type=jnp.float32)
            + jnp.dot(p2_ref[...], wfc_ref[_D:2 * _D, :],
                      preferred_element_type=jnp.float32)
            + bfc_ref[...])
        m = jnp.max(pooled, axis=-1, keepdims=True)
        lse = jnp.log(jnp.sum(jnp.exp(pooled - m), axis=-1, keepdims=True)) + m
        o_ref[...] = pooled - lse


def _final_tc(x1, a0, a1, w1, b1, w2, b2, batch3, wfc, bfc):
    row_spec = pl.BlockSpec((_BR, _D), lambda i: (i, 0))
    w_spec = pl.BlockSpec((_D, _D), lambda i: (0, 0))
    b_spec = pl.BlockSpec((1, _D), lambda i: (0, 0))
    return pl.pallas_call(
        _final_body,
        grid=(_NB,),
        in_specs=[
            row_spec, row_spec, row_spec, w_spec, b_spec, w_spec, b_spec,
            pl.BlockSpec((1, 1, _BR), lambda i: (i, 0, 0)),
            pl.BlockSpec((2 * _D, _D), lambda i: (0, 0)),
            b_spec,
        ],
        out_specs=pl.BlockSpec((_G, _D), lambda i: (0, 0)),
        out_shape=jax.ShapeDtypeStruct((_G, _D), jnp.float32),
        scratch_shapes=[
            pltpu.VMEM((_G, _D), jnp.float32),
            pltpu.VMEM((_G, _D), jnp.float32),
        ],
    )(x1, a0, a1, w1, b1, w2, b2, batch3, wfc, bfc)


def kernel(x, edge_index, batch, W1_0, b1_0, W2_0, b2_0,
           W1_1, b1_1, W2_1, b2_1, Wfc, bfc):
    srcp = edge_index[0].reshape(_NW * _NCHUNK, _CH)
    dstp = edge_index[1].reshape(_NW * _NCHUNK, _CH)
    zero_rows = jnp.zeros((_N + _PADROWS, _D), jnp.float32)

    b1_0r = b1_0.reshape(1, _D)
    b2_0r = b2_0.reshape(1, _D)
    b1_1r = b1_1.reshape(1, _D)
    b2_1r = b2_1.reshape(1, _D)
    bfcr = bfc.reshape(1, _D)
    batch3 = batch.reshape(_NB, 1, _BR)

    agg1 = _spmm_call(x, srcp, dstp, zero_rows)
    x1 = _mlp_tc(x, agg1[0], agg1[1], W1_0, b1_0r, W2_0, b2_0r)
    agg2 = _spmm_call(x1, srcp, dstp, zero_rows)
    out = _final_tc(x1, agg2[0], agg2[1], W1_1, b1_1r, W2_1, b2_1r,
                    batch3, Wfc, bfcr)
    return out


# speedup vs baseline: 10.0250x; 1.0224x over previous
import functools

import jax
import jax.numpy as jnp
from jax import lax
from jax.experimental import pallas as pl
from jax.experimental.pallas import tpu as pltpu
from jax.experimental.pallas import tpu_sc as plsc

_N = 10000
_E = 320000
_D = 128
_G = 64
_NC = 2
_NS = 16
_NW = _NC * _NS
_CH = 50
_EPT = 10000
_RPT = 624
_LAST = _N - _RPT * (_NS - 1)
_PADROWS = 16


_NCHUNK = _EPT // _CH
_NBUF = 4
_SEG = 40
_NSEG = _NCHUNK // _SEG


def _spmm_body(x_hbm, src_hbm, dst_hbm, zero_hbm, out_hbm,
               sidx, didx, rows, agg, semg, sems):
    cid = lax.axis_index("c")
    sid = lax.axis_index("s")
    wid = cid * _NS + sid
    base = sid * _RPT

    @pl.when(sid < _NS - 1)
    def _():
        pltpu.sync_copy(zero_hbm.at[pl.ds(base, _RPT)],
                        agg.at[pl.ds(base, _RPT)])

    @pl.when(sid == _NS - 1)
    def _():
        pltpu.sync_copy(zero_hbm.at[pl.ds(_RPT * (_NS - 1), _LAST + _PADROWS)],
                        agg.at[pl.ds(_RPT * (_NS - 1), _LAST + _PADROWS)])

    plsc.subcore_barrier()

    def gather(k, b):
        pltpu.async_copy(x_hbm.at[sidx.at[k]], rows[b], semg[b])

    def gather_wait(b):
        pltpu.make_async_copy(x_hbm.at[sidx.at[0]], rows[b], semg[b]).wait()

    def scatter(k, b):
        pltpu.async_copy(rows[b], agg.at[didx.at[k]], sems[b], add=True)

    def scatter_wait(b):
        pltpu.make_async_copy(rows[b], agg.at[didx.at[0]], sems[b]).wait()

    _NITER = _SEG // _NBUF

    for seg in range(_NSEG):
        segbase = wid * _NCHUNK + seg * _SEG
        pltpu.sync_copy(src_hbm.at[pl.ds(segbase, _SEG)], sidx)
        pltpu.sync_copy(dst_hbm.at[pl.ds(segbase, _SEG)], didx)

        for b in range(_NBUF):
            gather(b, b)

        def step(i, carry):
            k0 = i * _NBUF
            for b in range(_NBUF):
                gather_wait(b)
                scatter(k0 + b, b)
            for b in range(_NBUF):
                @pl.when(i < _NITER - 1)
                def _(b=b):
                    scatter_wait(b)
                    gather(k0 + _NBUF + b, b)
            return carry

        lax.fori_loop(0, _NITER, step, 0)
        for b in range(_NBUF):
            scatter_wait(b)

    plsc.subcore_barrier()

    @pl.when(jnp.logical_and(cid == 0, sid < _NS - 1))
    def _():
        pltpu.sync_copy(agg.at[pl.ds(base, _RPT)],
                        out_hbm.at[0].at[pl.ds(base, _RPT)])

    @pl.when(jnp.logical_and(cid == 0, sid == _NS - 1))
    def _():
        pltpu.sync_copy(agg.at[pl.ds(_RPT * (_NS - 1), _LAST)],
                        out_hbm.at[0].at[pl.ds(_RPT * (_NS - 1), _LAST)])

    @pl.when(jnp.logical_and(cid == 1, sid < _NS - 1))
    def _():
        pltpu.sync_copy(agg.at[pl.ds(base, _RPT)],
                        out_hbm.at[1].at[pl.ds(base, _RPT)])

    @pl.when(jnp.logical_and(cid == 1, sid == _NS - 1))
    def _():
        pltpu.sync_copy(agg.at[pl.ds(_RPT * (_NS - 1), _LAST)],
                        out_hbm.at[1].at[pl.ds(_RPT * (_NS - 1), _LAST)])


@functools.cache
def _get_spmm_call():
    return pl.kernel(
        _spmm_body,
        out_type=jax.ShapeDtypeStruct((_NC, _N, _D), jnp.float32),
        mesh=plsc.VectorSubcoreMesh(core_axis_name="c", subcore_axis_name="s",
                                    num_cores=_NC, num_subcores=_NS),
        scratch_types=[
            pltpu.VMEM((_SEG, _CH), jnp.int32),
            pltpu.VMEM((_SEG, _CH), jnp.int32),
            [pltpu.VMEM((_CH, _D), jnp.float32) for _ in range(_NBUF)],
            pltpu.VMEM_SHARED((_N + _PADROWS, _D), jnp.float32),
            [pltpu.SemaphoreType.DMA for _ in range(_NBUF)],
            [pltpu.SemaphoreType.DMA for _ in range(_NBUF)],
        ],
    )


def _spmm_call(x, srcp, dstp, zero_rows):
    return _get_spmm_call()(x, srcp, dstp, zero_rows)


def _mlp_body(x_ref, a0_ref, a1_ref, w1_ref, b1_ref, w2_ref, b2_ref, o_ref):
    h = x_ref[...] + a0_ref[...] + a1_ref[...]
    h = jnp.maximum(
        jnp.dot(h, w1_ref[...], preferred_element_type=jnp.float32)
        + b1_ref[...], 0.0)
    h = jnp.maximum(
        jnp.dot(h, w2_ref[...], preferred_element_type=jnp.float32)
        + b2_ref[...], 0.0)
    o_ref[...] = h


_BR = 2000
_NB = _N // _BR


def _mlp_tc(x, a0, a1, w1, b1, w2, b2):
    row_spec = pl.BlockSpec((_BR, _D), lambda i: (i, 0))
    w_spec = pl.BlockSpec((_D, _D), lambda i: (0, 0))
    b_spec = pl.BlockSpec((1, _D), lambda i: (0, 0))
    return pl.pallas_call(
        _mlp_body,
        grid=(_NB,),
        in_specs=[row_spec, row_spec, row_spec, w_spec, b_spec, w_spec, b_spec],
        out_specs=row_spec,
        out_shape=jax.ShapeDtypeStruct((_N, _D), jnp.float32),
    )(x, a0, a1, w1, b1, w2, b2)


def _final_body(x_ref, a0_ref, a1_ref, w1_ref, b1_ref, w2_ref, b2_ref,
                bt_ref, wfc_ref, bfc_ref, o_ref, p1_ref, p2_ref):
    i = pl.program_id(0)

    @pl.when(i == 0)
    def _():
        p1_ref[...] = jnp.zeros_like(p1_ref)
        p2_ref[...] = jnp.zeros_like(p2_ref)

    x1b = x_ref[...]
    h = x1b + a0_ref[...] + a1_ref[...]
    h = jnp.maximum(
        jnp.dot(h, w1_ref[...], preferred_element_type=jnp.float32)
        + b1_ref[...], 0.0)
    x2b = jnp.maximum(
        jnp.dot(h, w2_ref[...], preferred_element_type=jnp.float32)
        + b2_ref[...], 0.0)

    b2d = bt_ref[...].reshape(1, _BR)
    gids = lax.broadcasted_iota(jnp.int32, (_G, _BR), 0)
    pt = (gids == b2d).astype(jnp.float32)
    p1_ref[...] += jnp.dot(pt, x1b, preferred_element_type=jnp.float32)
    p2_ref[...] += jnp.dot(pt, x2b, preferred_element_type=jnp.float32)

    @pl.when(i == _NB - 1)
    def _():
        pooled = (
            jnp.dot(p1_ref[...], wfc_ref[0:_D, :],
                    preferred_element_type=jnp.float32)
            + jnp.dot(p2_ref[...], wfc_ref[_D:2 * _D, :],
                      preferred_element_type=jnp.float32)
            + bfc_ref[...])
        m = jnp.max(pooled, axis=-1, keepdims=True)
        lse = jnp.log(jnp.sum(jnp.exp(pooled - m), axis=-1, keepdims=True)) + m
        o_ref[...] = pooled - lse


def _final_tc(x1, a0, a1, w1, b1, w2, b2, batch3, wfc, bfc):
    row_spec = pl.BlockSpec((_BR, _D), lambda i: (i, 0))
    w_spec = pl.BlockSpec((_D, _D), lambda i: (0, 0))
    b_spec = pl.BlockSpec((1, _D), lambda i: (0, 0))
    return pl.pallas_call(
        _final_body,
        grid=(_NB,),
        in_specs=[
            row_spec, row_spec, row_spec, w_spec, b_spec, w_spec, b_spec,
            pl.BlockSpec((1, 1, _BR), lambda i: (i, 0, 0)),
            pl.BlockSpec((2 * _D, _D), lambda i: (0, 0)),
            b_spec,
        ],
        out_specs=pl.BlockSpec((_G, _D), lambda i: (0, 0)),
        out_shape=jax.ShapeDtypeStruct((_G, _D), jnp.float32),
        scratch_shapes=[
            pltpu.VMEM((_G, _D), jnp.float32),
            pltpu.VMEM((_G, _D), jnp.float32),
        ],
    )(x1, a0, a1, w1, b1, w2, b2, batch3, wfc, bfc)


def kernel(x, edge_index, batch, W1_0, b1_0, W2_0, b2_0,
           W1_1, b1_1, W2_1, b2_1, Wfc, bfc):
    srcp = edge_index[0].reshape(_NW * _NCHUNK, _CH)
    dstp = edge_index[1].reshape(_NW * _NCHUNK, _CH)
    zero_rows = jnp.zeros((_N + _PADROWS, _D), jnp.float32)

    b1_0r = b1_0.reshape(1, _D)
    b2_0r = b2_0.reshape(1, _D)
    b1_1r = b1_1.reshape(1, _D)
    b2_1r = b2_1.reshape(1, _D)
    bfcr = bfc.reshape(1, _D)
    batch3 = batch.reshape(_NB, 1, _BR)

    agg1 = _spmm_call(x, srcp, dstp, zero_rows)
    x1 = _mlp_tc(x, agg1[0], agg1[1], W1_0, b1_0r, W2_0, b2_0r)
    agg2 = _spmm_call(x1, srcp, dstp, zero_rows)
    out = _final_tc(x1, agg2[0], agg2[1], W1_1, b1_1r, W2_1, b2_1r,
                    batch3, Wfc, bfcr)
    return out
